# baseline (speedup 1.0000x reference)
import functools

import jax
import jax.numpy as jnp
from jax import lax
from jax.experimental import pallas as pl
from jax.experimental.pallas import tpu as pltpu

R = 2048
C = 128
RBITS = 11
LOGN = 18


def _lovasz_body(l_ref, t_ref, o_ref, keys):
    row = lax.broadcasted_iota(jnp.int32, (R, C), 0)
    lane = lax.broadcasted_iota(jnp.int32, (R, C), 1)
    idx = row + R * lane

    l = l_ref[0]
    t = t_ref[0]
    tf = t.astype(jnp.float32)
    err = 1.0 - l * (2.0 * tf - 1.0)
    relu_err = jnp.maximum(err, 0.0)
    bits = lax.bitcast_convert_type(relu_err, jnp.int32)
    keys[...] = (bits & jnp.int32(-2)) | t
    gts = jnp.sum(tf)

    def stage(j, k):
        x = keys[...]
        sj = jnp.left_shift(jnp.int32(1), j)
        bitj = (idx & sj) != 0
        dirbit = (idx & jnp.left_shift(jnp.int32(1), k)) != 0

        def row_partner():
            up = pltpu.roll(x, R - sj, axis=0)
            dn = pltpu.roll(x, sj, axis=0)
            return jnp.where(bitj, dn, up)

        def lane_partner():
            sl = jnp.right_shift(sj, RBITS)
            up = pltpu.roll(x, C - sl, axis=1)
            dn = pltpu.roll(x, sl, axis=1)
            return jnp.where(bitj, dn, up)

        partner = lax.cond(j < RBITS, row_partner, lane_partner)
        mn = jnp.minimum(x, partner)
        mx = jnp.maximum(x, partner)
        keys[...] = jnp.where(bitj == dirbit, mx, mn)

    def outer(k, carry):
        def inner(tt, carry2):
            stage(k - 1 - tt, k)
            return carry2

        return lax.fori_loop(0, k, inner, carry)

    lax.fori_loop(1, LOGN + 1, outer, 0)

    ks = keys[...]
    gt_sorted = (ks & 1).astype(jnp.float32)
    err_sorted = lax.bitcast_convert_type(ks & jnp.int32(-2), jnp.float32)

    cs = gt_sorted
    for m in range(RBITS):
        sh = 1 << m
        cs = cs + jnp.where(row >= sh, pltpu.roll(cs, sh, axis=0), 0.0)
    col_tot = jnp.sum(gt_sorted, axis=0, keepdims=True)
    lane1 = lax.broadcasted_iota(jnp.int32, (1, C), 1)
    lt = col_tot
    for m in range(LOGN - RBITS):
        sh = 1 << m
        lt = lt + jnp.where(lane1 >= sh, pltpu.roll(lt, sh, axis=1), 0.0)
    cs = cs + (lt - col_tot)

    pos = (idx + 1).astype(jnp.float32)
    jac = 1.0 - (gts - cs) / (gts + pos - cs)
    p1 = pltpu.roll(jac, 1, axis=0)
    prev = jnp.where(row == 0, pltpu.roll(p1, 1, axis=1), p1)
    prev = jnp.where((row == 0) & (lane == 0), 0.0, prev)
    o_ref[...] = jnp.full((1, 8, 128), jnp.sum(err_sorted * (jac - prev)),
                          dtype=jnp.float32)


@jax.jit
def kernel(logits, targets):
    b = logits.shape[0]
    lf = logits[:, 0].reshape(b, R, C)
    tg = targets.reshape(b, R, C)
    losses = pl.pallas_call(
        _lovasz_body,
        grid=(b,),
        in_specs=[
            pl.BlockSpec((1, R, C), lambda i: (i, 0, 0)),
            pl.BlockSpec((1, R, C), lambda i: (i, 0, 0)),
        ],
        out_specs=pl.BlockSpec((1, 8, 128), lambda i: (i, 0, 0)),
        out_shape=jax.ShapeDtypeStruct((b, 8, 128), jnp.float32),
        scratch_shapes=[pltpu.VMEM((R, C), jnp.int32)],
        compiler_params=pltpu.CompilerParams(
            dimension_semantics=("arbitrary",),
        ),
    )(lf, tg)
    return jnp.mean(losses[:, 0, 0])
